# Initial kernel scaffold; baseline (speedup 1.0000x reference)
import functools

import jax
import jax.numpy as jnp
from jax import lax
from jax.experimental import pallas as pl
from jax.experimental.pallas import tpu as pltpu
from jax.experimental.pallas import tpu_sc as plsc

B = 8
T = 8192
D = 1024
H = 512
L = 32
NC = 256
KEEP = 128
TBLK = 512
NT = T // TBLK
CPB = TBLK // L

OUT_ROWS = B * KEEP * L
ROWS_PER_TILE = OUT_ROWS // 16
CH = 32
NB = ROWS_PER_TILE // CH


def _score_body(k_ref, v_ref, w1_ref, b1_ref, w2t_ref, out_ref):
    x = (k_ref[0] + v_ref[0]) * 0.5
    h = jnp.dot(x, w1_ref[...])
    h = jnp.maximum(h + b1_ref[...], 0.0)
    s_row = lax.dot_general(w2t_ref[...], h,
                            dimension_numbers=(((1,), (1,)), ((), ())))
    tok = lax.broadcasted_iota(jnp.int32, (TBLK, CPB), 0)
    chk = lax.broadcasted_iota(jnp.int32, (TBLK, CPB), 1)
    m2 = (tok // L == chk).astype(jnp.float32)
    out_ref[...] = jnp.dot(s_row, m2, precision=lax.Precision.HIGHEST)


def _topk_body(cs_ref, out_ref):
    s = cs_ref[...]
    si = s[:, None, :]
    sj = s[:, :, None]
    ii = lax.broadcasted_iota(jnp.int32, (B, NC, NC), 2)
    jj = lax.broadcasted_iota(jnp.int32, (B, NC, NC), 1)
    gt = (sj > si).astype(jnp.float32)
    eq = ((sj == si) & (jj < ii)).astype(jnp.float32)
    cnt = jnp.sum(gt + eq, axis=1)
    keepf = (cnt < float(KEEP)).astype(jnp.float32)
    lt = (lax.broadcasted_iota(jnp.int32, (NC, NC), 0)
          < lax.broadcasted_iota(jnp.int32, (NC, NC), 1)).astype(jnp.float32)
    rank = jnp.dot(keepf, lt, precision=lax.Precision.HIGHEST)
    piota = lax.broadcasted_iota(jnp.float32, (B, KEEP, NC), 1)
    slot = ((rank[:, None, :] == piota)
            & (keepf[:, None, :] > 0.0)).astype(jnp.float32)
    ival = lax.broadcasted_iota(jnp.float32, (B, KEEP, NC), 2)
    chunk3 = jnp.sum(slot * ival, axis=2, keepdims=True)
    l_io = lax.broadcasted_iota(jnp.float32, (B, KEEP, L), 2)
    b_io = lax.broadcasted_iota(jnp.float32, (B, KEEP, L), 0)
    tok = b_io * float(T) + chunk3 * float(L) + l_io
    out_ref[...] = tok.astype(jnp.int32)


def _gather_body(keys_ref, vals_ref, idx_ref, outk_ref, outv_ref,
                 idxv, buf, sem):
    c = lax.axis_index("c")
    s = lax.axis_index("s")
    base = s * ROWS_PER_TILE
    pltpu.sync_copy(idx_ref.at[pl.ds(base, ROWS_PER_TILE)], idxv)

    def body(nb, carry):
        g = nb * CH
        gi = idxv.at[pl.ds(g, CH)]

        @pl.when(c == 0)
        def _():
            pltpu.async_copy(keys_ref.at[gi], buf, sem).wait()
            pltpu.sync_copy(buf, outk_ref.at[pl.ds(base + g, CH)])

        @pl.when(c == 1)
        def _():
            pltpu.async_copy(vals_ref.at[gi], buf, sem).wait()
            pltpu.sync_copy(buf, outv_ref.at[pl.ds(base + g, CH)])

        return carry

    lax.fori_loop(0, NB, body, 0)


def _scores(keys, values, W1, b1, W2):
    return pl.pallas_call(
        _score_body,
        grid=(B, NT),
        in_specs=[
            pl.BlockSpec((1, TBLK, D), lambda b, t: (b, t, 0)),
            pl.BlockSpec((1, TBLK, D), lambda b, t: (b, t, 0)),
            pl.BlockSpec((D, H), lambda b, t: (0, 0)),
            pl.BlockSpec((1, H), lambda b, t: (0, 0)),
            pl.BlockSpec((1, H), lambda b, t: (0, 0)),
        ],
        out_specs=pl.BlockSpec((1, CPB), lambda b, t: (b, t)),
        out_shape=jax.ShapeDtypeStruct((B, NC), jnp.float32),
    )(keys, values, W1, b1.reshape(1, H), W2.reshape(1, H))


def _topk_rows(chunk_scores):
    return pl.pallas_call(
        _topk_body,
        out_shape=jax.ShapeDtypeStruct((B, KEEP, L), jnp.int32),
    )(chunk_scores)


@functools.partial(
    pl.kernel,
    mesh=plsc.VectorSubcoreMesh(core_axis_name="c", subcore_axis_name="s"),
    out_type=(jax.ShapeDtypeStruct((OUT_ROWS, D), jnp.float32),
              jax.ShapeDtypeStruct((OUT_ROWS, D), jnp.float32)),
    scratch_types=[
        pltpu.VMEM((ROWS_PER_TILE,), jnp.int32),
        pltpu.VMEM((CH, D), jnp.float32),
        pltpu.SemaphoreType.DMA,
    ],
)
def _gather(keys2d, vals2d, idx_flat, outk, outv, idxv, buf, sem):
    _gather_body(keys2d, vals2d, idx_flat, outk, outv, idxv, buf, sem)


def kernel(keys, values, W1, b1, W2, b2):
    del b2
    cs = _scores(keys, values, W1, b1, W2)
    rows = _topk_rows(cs).reshape(OUT_ROWS)
    outk, outv = _gather(keys.reshape(B * T, D), values.reshape(B * T, D), rows)
    return (outk.reshape(B, KEEP * L, D), outv.reshape(B, KEEP * L, D))


# speedup vs baseline: 1.4618x; 1.4618x over previous
import functools

import jax
import jax.numpy as jnp
from jax import lax
from jax.experimental import pallas as pl
from jax.experimental.pallas import tpu as pltpu
from jax.experimental.pallas import tpu_sc as plsc

B = 8
T = 8192
D = 1024
H = 512
L = 32
NC = 256
KEEP = 128
TBLK = 512
NT = T // TBLK
CPB = TBLK // L

OUT_ROWS = B * KEEP * L
ROWS_PER_TILE = OUT_ROWS // 16
CH = 32
NB = ROWS_PER_TILE // CH


def _score_body(k_ref, v_ref, w1_ref, b1_ref, w2t_ref, out_ref):
    x = (k_ref[0] + v_ref[0]) * 0.5
    h = jnp.dot(x, w1_ref[...])
    h = jnp.maximum(h + b1_ref[...], 0.0)
    s_row = lax.dot_general(w2t_ref[...], h,
                            dimension_numbers=(((1,), (1,)), ((), ())))
    tok = lax.broadcasted_iota(jnp.int32, (TBLK, CPB), 0)
    chk = lax.broadcasted_iota(jnp.int32, (TBLK, CPB), 1)
    m2 = (tok // L == chk).astype(jnp.float32)
    c_row = jnp.dot(s_row, m2, precision=lax.Precision.HIGHEST)
    out_ref[...] = c_row.reshape(1, 1, 1, CPB)


def _topk_body(cs_ref, out_ref):
    s = cs_ref[...]
    si = s[:, None, :]
    sj = s[:, :, None]
    ii = lax.broadcasted_iota(jnp.int32, (B, NC, NC), 2)
    jj = lax.broadcasted_iota(jnp.int32, (B, NC, NC), 1)
    gt = (sj > si).astype(jnp.float32)
    eq = ((sj == si) & (jj < ii)).astype(jnp.float32)
    cnt = jnp.sum(gt + eq, axis=1)
    keepf = (cnt < float(KEEP)).astype(jnp.float32)
    lt = (lax.broadcasted_iota(jnp.int32, (NC, NC), 0)
          < lax.broadcasted_iota(jnp.int32, (NC, NC), 1)).astype(jnp.float32)
    rank = jnp.dot(keepf, lt, precision=lax.Precision.HIGHEST)
    ranki = rank.astype(jnp.int32)
    piota = lax.broadcasted_iota(jnp.int32, (B, KEEP, NC), 1)
    slot = ((ranki[:, None, :] == piota)
            & (keepf[:, None, :] > 0.0)).astype(jnp.int32)
    ival = lax.broadcasted_iota(jnp.int32, (B, KEEP, NC), 2)
    chunk3 = jnp.sum(slot * ival, axis=2, keepdims=True)
    l_io = lax.broadcasted_iota(jnp.int32, (B, KEEP, L), 2)
    b_io = lax.broadcasted_iota(jnp.int32, (B, KEEP, L), 0)
    out_ref[...] = b_io * T + chunk3 * L + l_io


def _gather_body(keys_ref, vals_ref, idx_ref, outk_ref, outv_ref,
                 idxv, buf, sem):
    c = lax.axis_index("c")
    s = lax.axis_index("s")
    base = s * ROWS_PER_TILE
    pltpu.sync_copy(idx_ref.at[pl.ds(base, ROWS_PER_TILE)], idxv)

    def body(nb, carry):
        g = nb * CH
        gi = idxv.at[pl.ds(g, CH)]

        @pl.when(c == 0)
        def _():
            pltpu.async_copy(keys_ref.at[gi], buf, sem).wait()
            pltpu.sync_copy(buf, outk_ref.at[pl.ds(base + g, CH)])

        @pl.when(c == 1)
        def _():
            pltpu.async_copy(vals_ref.at[gi], buf, sem).wait()
            pltpu.sync_copy(buf, outv_ref.at[pl.ds(base + g, CH)])

        return carry

    lax.fori_loop(0, NB, body, 0)


def _scores(keys, values, W1, b1, W2):
    return pl.pallas_call(
        _score_body,
        grid=(B, NT),
        in_specs=[
            pl.BlockSpec((1, TBLK, D), lambda b, t: (b, t, 0)),
            pl.BlockSpec((1, TBLK, D), lambda b, t: (b, t, 0)),
            pl.BlockSpec((D, H), lambda b, t: (0, 0)),
            pl.BlockSpec((1, H), lambda b, t: (0, 0)),
            pl.BlockSpec((1, H), lambda b, t: (0, 0)),
        ],
        out_specs=pl.BlockSpec((1, 1, 1, CPB), lambda b, t: (b, t, 0, 0)),
        out_shape=jax.ShapeDtypeStruct((B, NT, 1, CPB), jnp.float32),
    )(keys, values, W1, b1.reshape(1, H), W2.reshape(1, H)).reshape(B, NC)


def _topk_rows(chunk_scores):
    return pl.pallas_call(
        _topk_body,
        out_shape=jax.ShapeDtypeStruct((B, KEEP, L), jnp.int32),
    )(chunk_scores)


@functools.cache
def _gather_kernel():
    return pl.kernel(
        _gather_body,
        mesh=plsc.VectorSubcoreMesh(core_axis_name="c", subcore_axis_name="s"),
        out_type=(jax.ShapeDtypeStruct((OUT_ROWS, D), jnp.float32),
                  jax.ShapeDtypeStruct((OUT_ROWS, D), jnp.float32)),
        scratch_types=[
            pltpu.VMEM((ROWS_PER_TILE,), jnp.int32),
            pltpu.VMEM((CH, D), jnp.float32),
            pltpu.SemaphoreType.DMA,
        ],
    )


def kernel(keys, values, W1, b1, W2, b2):
    del b2
    cs = _scores(keys, values, W1, b1, W2)
    rows = _topk_rows(cs).reshape(OUT_ROWS)
    outk, outv = _gather_kernel()(
        keys.reshape(B * T, D), values.reshape(B * T, D), rows)
    return (outk.reshape(B, KEEP * L, D), outv.reshape(B, KEEP * L, D))


# speedup vs baseline: 1.5888x; 1.0869x over previous
import functools

import jax
import jax.numpy as jnp
from jax import lax
from jax.experimental import pallas as pl
from jax.experimental.pallas import tpu as pltpu
from jax.experimental.pallas import tpu_sc as plsc

B = 8
T = 8192
D = 1024
H = 512
L = 32
NC = 256
KEEP = 128
TBLK = 512
NT = T // TBLK
CPB = TBLK // L

OUT_ROWS = B * KEEP * L
ROWS_PER_TILE = OUT_ROWS // 16
CH = 32
NB = ROWS_PER_TILE // CH


def _score_body(k_ref, v_ref, w1_ref, b1_ref, w2t_ref, out_ref):
    x = (k_ref[0] + v_ref[0]) * 0.5
    h = jnp.dot(x, w1_ref[...])
    h = jnp.maximum(h + b1_ref[...], 0.0)
    s_row = lax.dot_general(w2t_ref[...], h,
                            dimension_numbers=(((1,), (1,)), ((), ())))
    tok = lax.broadcasted_iota(jnp.int32, (TBLK, CPB), 0)
    chk = lax.broadcasted_iota(jnp.int32, (TBLK, CPB), 1)
    m2 = (tok // L == chk).astype(jnp.float32)
    c_row = jnp.dot(s_row, m2, precision=lax.Precision.HIGHEST)
    out_ref[...] = c_row.reshape(1, 1, 1, CPB)


def _topk_body(cs_ref, out_ref):
    s = cs_ref[...]
    si = s[:, None, :]
    sj = s[:, :, None]
    ii = lax.broadcasted_iota(jnp.int32, (B, NC, NC), 2)
    jj = lax.broadcasted_iota(jnp.int32, (B, NC, NC), 1)
    gt = (sj > si).astype(jnp.float32)
    eq = ((sj == si) & (jj < ii)).astype(jnp.float32)
    cnt = jnp.sum(gt + eq, axis=1)
    keepf = (cnt < float(KEEP)).astype(jnp.float32)
    lt = (lax.broadcasted_iota(jnp.int32, (NC, NC), 0)
          < lax.broadcasted_iota(jnp.int32, (NC, NC), 1)).astype(jnp.float32)
    rank = jnp.dot(keepf, lt, precision=lax.Precision.HIGHEST)
    ranki = rank.astype(jnp.int32)
    piota = lax.broadcasted_iota(jnp.int32, (B, KEEP, NC), 1)
    slot = ((ranki[:, None, :] == piota)
            & (keepf[:, None, :] > 0.0)).astype(jnp.int32)
    ival = lax.broadcasted_iota(jnp.int32, (B, KEEP, NC), 2)
    chunk3 = jnp.sum(slot * ival, axis=2, keepdims=True)
    l_io = lax.broadcasted_iota(jnp.int32, (B, KEEP, L), 2)
    b_io = lax.broadcasted_iota(jnp.int32, (B, KEEP, L), 0)
    out_ref[...] = b_io * T + chunk3 * L + l_io


def _gather_body(keys_ref, vals_ref, idx_ref, outk_ref, outv_ref,
                 idxv, buf0, buf1, sem0, sem1):
    c = lax.axis_index("c")
    s = lax.axis_index("s")
    base = s * ROWS_PER_TILE
    pltpu.sync_copy(idx_ref.at[pl.ds(base, ROWS_PER_TILE)], idxv)

    def run(table, out):
        pltpu.async_copy(table.at[idxv.at[pl.ds(0, CH)]], buf0, sem0)

        def body(i, carry):
            g0 = (2 * i) * CH
            g1 = g0 + CH
            g2 = g1 + CH
            pltpu.make_async_copy(table.at[pl.ds(0, CH)], buf0, sem0).wait()
            pltpu.async_copy(table.at[idxv.at[pl.ds(g1, CH)]], buf1, sem1)
            pltpu.sync_copy(buf0, out.at[pl.ds(base + g0, CH)])
            pltpu.make_async_copy(table.at[pl.ds(0, CH)], buf1, sem1).wait()

            @pl.when(i < NB // 2 - 1)
            def _():
                pltpu.async_copy(table.at[idxv.at[pl.ds(g2, CH)]], buf0, sem0)

            pltpu.sync_copy(buf1, out.at[pl.ds(base + g1, CH)])
            return carry

        lax.fori_loop(0, NB // 2, body, 0)

    @pl.when(c == 0)
    def _():
        run(keys_ref, outk_ref)

    @pl.when(c == 1)
    def _():
        run(vals_ref, outv_ref)


def _scores(keys, values, W1, b1, W2):
    return pl.pallas_call(
        _score_body,
        grid=(B, NT),
        in_specs=[
            pl.BlockSpec((1, TBLK, D), lambda b, t: (b, t, 0)),
            pl.BlockSpec((1, TBLK, D), lambda b, t: (b, t, 0)),
            pl.BlockSpec((D, H), lambda b, t: (0, 0)),
            pl.BlockSpec((1, H), lambda b, t: (0, 0)),
            pl.BlockSpec((1, H), lambda b, t: (0, 0)),
        ],
        out_specs=pl.BlockSpec((1, 1, 1, CPB), lambda b, t: (b, t, 0, 0)),
        out_shape=jax.ShapeDtypeStruct((B, NT, 1, CPB), jnp.float32),
    )(keys, values, W1, b1.reshape(1, H), W2.reshape(1, H)).reshape(B, NC)


def _topk_rows(chunk_scores):
    return pl.pallas_call(
        _topk_body,
        out_shape=jax.ShapeDtypeStruct((B, KEEP, L), jnp.int32),
    )(chunk_scores)


@functools.cache
def _gather_kernel():
    return pl.kernel(
        _gather_body,
        mesh=plsc.VectorSubcoreMesh(core_axis_name="c", subcore_axis_name="s"),
        out_type=(jax.ShapeDtypeStruct((OUT_ROWS, D), jnp.float32),
                  jax.ShapeDtypeStruct((OUT_ROWS, D), jnp.float32)),
        scratch_types=[
            pltpu.VMEM((ROWS_PER_TILE,), jnp.int32),
            pltpu.VMEM((CH, D), jnp.float32),
            pltpu.VMEM((CH, D), jnp.float32),
            pltpu.SemaphoreType.DMA,
            pltpu.SemaphoreType.DMA,
        ],
    )


def kernel(keys, values, W1, b1, W2, b2):
    del b2
    cs = _scores(keys, values, W1, b1, W2)
    rows = _topk_rows(cs).reshape(OUT_ROWS)
    outk, outv = _gather_kernel()(
        keys.reshape(B * T, D), values.reshape(B * T, D), rows)
    return (outk.reshape(B, KEEP * L, D), outv.reshape(B, KEEP * L, D))
